# baseline (speedup 1.0000x reference)
import functools

import jax
import jax.numpy as jnp
from jax import lax
from jax.experimental import pallas as pl
from jax.experimental.pallas import tpu as pltpu
from jax.experimental.pallas import tpu_sc as plsc

NC = 2
NS = 16
L = 16
NW = NC * NS

DIM = 64

CHUNK = 256
EPS2 = 1e-16
THRESH = 0.8
IMAX = 2**31 - 1


def _sqrt_vec(x):
    xi = plsc.bitcast(x, jnp.int32)
    r = plsc.bitcast(jnp.int32(0x5F3759DF) - (xi >> 1), jnp.float32)
    for _ in range(4):
        r = r * (1.5 - 0.5 * x * r * r)
    return x * r


SPW = 2048
PW = SPW // NS


def _make_sc_call(n_rows):
    per_core = n_rows // NC
    K = per_core // SPW
    assert K * SPW == per_core and K >= 3 and K % 2 == 0
    trans_pairs = (K - 2) // 2

    mesh = plsc.VectorSubcoreMesh(
        core_axis_name="c", subcore_axis_name="s",
        num_cores=NC, num_subcores=NS)

    lane = lambda: lax.broadcasted_iota(jnp.int32, (L,), 0)

    scratch = [
        pltpu.VMEM((DIM, PW), jnp.float32),
        pltpu.VMEM((DIM, PW), jnp.float32),
        pltpu.VMEM((DIM,), jnp.float32),
        pltpu.VMEM((L,), jnp.float32),
        pltpu.VMEM((L,), jnp.int32),
        pltpu.VMEM((NS * L,), jnp.float32),
        pltpu.VMEM((NS * L,), jnp.int32),
        pltpu.VMEM((L,), jnp.float32),
        pltpu.VMEM((L,), jnp.int32),
        pltpu.VMEM_SHARED((DIM, SPW), jnp.float32),
        pltpu.VMEM_SHARED((DIM, SPW), jnp.float32),
        pltpu.VMEM_SHARED((NS * L,), jnp.float32),
        pltpu.VMEM_SHARED((NS * L,), jnp.int32),
        pltpu.SemaphoreType.DMA,
        pltpu.SemaphoreType.DMA,
        pltpu.SemaphoreType.DMA,
        pltpu.SemaphoreType.DMA,
    ]

    @functools.partial(
        pl.kernel,
        out_type=(
            jax.ShapeDtypeStruct((NC * L,), jnp.float32),
            jax.ShapeDtypeStruct((NC * L,), jnp.float32),
            jax.ShapeDtypeStruct((NC * L,), jnp.int32),
            jax.ShapeDtypeStruct((NC * L,), jnp.int32),
        ),
        mesh=mesh,
        scratch_types=scratch,
        compiler_params=pltpu.CompilerParams(
            needs_layout_passes=False, skip_device_barrier=True),
    )
    def sc_call(nm_hbm, emt_hbm, out_sim, out_m, out_idx, out_risk,
                tb0, tb1, nm_v, cm_v, ci_v, am_v, ai_v,
                sim_v, risk_v, sp0, sp1, spm_m, spm_i,
                semd0, semd1, sems0, sems1):
        cid = lax.axis_index("c")
        sid = lax.axis_index("s")
        core_base = cid * per_core

        pltpu.sync_copy(nm_hbm, nm_v)
        nm_s = []
        for c0 in range(0, DIM, L):
            v = nm_v[pl.ds(c0, L)]
            nm_s.extend(v[j] for j in range(L))

        def dma_src(s):
            return emt_hbm.at[:, pl.ds(core_base + s * SPW, SPW)]

        def start_dma(s, sp, sem):
            pltpu.async_copy(dma_src(s), sp, sem)

        def wait_dma(s, sp, sem):
            pltpu.make_async_copy(dma_src(s), sp, sem).wait()

        def start_stream(sp, tb, sem):
            pltpu.async_copy(sp.at[:, pl.ds(sid * PW, PW)], tb, sem)

        def wait_stream(sp, tb, sem):
            pltpu.make_async_copy(
                sp.at[:, pl.ds(sid * PW, PW)], tb, sem).wait()

        def process(buf, col0, n_groups, best):
            best_m, best_i = best
            d = [jnp.zeros((L,), jnp.float32) for _ in range(n_groups)]
            q = [jnp.zeros((L,), jnp.float32) for _ in range(n_groups)]
            for c in range(DIM):
                s = nm_s[c]
                for g in range(n_groups):
                    e = buf[c, pl.ds(g * L, L)]
                    d[g] = d[g] + e * s
                    q[g] = q[g] + e * e
            for g in range(n_groups):
                m = d[g] * jnp.abs(d[g]) / jnp.maximum(q[g], EPS2)
                idx = col0 + g * L + lane()
                better = m > best_m
                best_m = jnp.where(better, m, best_m)
                best_i = jnp.where(better, idx, best_i)
            return (best_m, best_i)

        best = (jnp.full((L,), -jnp.inf, jnp.float32),
                jnp.full((L,), IMAX, jnp.int32))

        slots = ((sp0, semd0, tb0, sems0), (sp1, semd1, tb1, sems1))

        @pl.when(sid == 0)
        def _():
            start_dma(0, sp0, semd0)
            start_dma(1, sp1, semd1)
            wait_dma(0, sp0, semd0)
        plsc.subcore_barrier()
        start_stream(sp0, tb0, sems0)

        def transition(s, b, best):
            sp_a, semd_a, tb_a, sems_a = slots[b]
            sp_b, semd_b, tb_b, sems_b = slots[1 - b]

            @pl.when(sid == 0)
            def _():
                wait_dma(s + 1, sp_b, semd_b)
            plsc.subcore_barrier()
            start_stream(sp_b, tb_b, sems_b)
            wait_stream(sp_a, tb_a, sems_a)
            plsc.subcore_barrier()

            @pl.when(sid == 0)
            def _():
                start_dma(s + 2, sp_a, semd_a)
            best = process(tb_a, core_base + s * SPW + sid * PW,
                           PW // L, best)
            return best

        def pair_body(t, best):
            best = transition(2 * t, 0, best)
            best = transition(2 * t + 1, 1, best)
            return best

        best = lax.fori_loop(0, trans_pairs, pair_body, best)

        s = K - 2
        sp_a, semd_a, tb_a, sems_a = slots[0]
        sp_b, semd_b, tb_b, sems_b = slots[1]

        @pl.when(sid == 0)
        def _():
            wait_dma(s + 1, sp_b, semd_b)
        plsc.subcore_barrier()
        start_stream(sp_b, tb_b, sems_b)
        wait_stream(sp_a, tb_a, sems_a)
        best = process(tb_a, core_base + s * SPW + sid * PW, PW // L, best)
        wait_stream(sp_b, tb_b, sems_b)
        best = process(tb_b, core_base + (s + 1) * SPW + sid * PW,
                       PW // L, best)

        best_m, best_i = best

        cm_v[...] = best_m
        ci_v[...] = best_i
        pltpu.sync_copy(cm_v, spm_m.at[pl.ds(sid * L, L)])
        pltpu.sync_copy(ci_v, spm_i.at[pl.ds(sid * L, L)])
        plsc.subcore_barrier()

        @pl.when(sid == 0)
        def _():
            pltpu.sync_copy(spm_m, am_v)
            pltpu.sync_copy(spm_i, ai_v)
            m = am_v[pl.ds(0, L)]
            i = ai_v[pl.ds(0, L)]
            for s in range(1, NS):
                mn = am_v[pl.ds(s * L, L)]
                iN = ai_v[pl.ds(s * L, L)]
                bet = (mn > m) | ((mn == m) & (iN < i))
                m = jnp.where(bet, mn, m)
                i = jnp.where(bet, iN, i)
            m_s = jnp.max(m)
            i_s = jnp.min(jnp.where(m == m_s, i, IMAX))

            nm2_s = jnp.float32(0.0)
            for c0 in range(0, DIM, L):
                v = nm_v[pl.ds(c0, L)]
                nm2_s = nm2_s + jnp.sum(v * v)

            mfin = jnp.full((L,), m_s, jnp.float32)
            s2 = jnp.abs(mfin) / jnp.maximum(
                jnp.full((L,), nm2_s, jnp.float32), EPS2)
            simv = jnp.sign(mfin) * _sqrt_vec(s2)

            sim_v[...] = simv
            cm_v[...] = mfin
            ci_v[...] = jnp.full((L,), i_s, jnp.int32)
            risk_v[...] = jnp.where(simv > THRESH, 1, 0).astype(jnp.int32)
            pltpu.sync_copy(sim_v, out_sim.at[pl.ds(cid * L, L)])
            pltpu.sync_copy(cm_v, out_m.at[pl.ds(cid * L, L)])
            pltpu.sync_copy(ci_v, out_idx.at[pl.ds(cid * L, L)])
            pltpu.sync_copy(risk_v, out_risk.at[pl.ds(cid * L, L)])

    return sc_call


BT = 16384


def _make_tc_call(n_rows, split):
    n_tc = n_rows - split
    grid = (n_tc + BT - 1) // BT
    assert split % BT == 0

    def body(nm_ref, emt_ref, out_ref, bm_ref, bi_ref):
        i = pl.program_id(0)
        e = emt_ref[...]
        nm = nm_ref[...]
        d = jnp.sum(e * nm, axis=0, keepdims=True)
        q = jnp.sum(e * e, axis=0, keepdims=True)
        m = d * jnp.abs(d) / jnp.maximum(q, EPS2)
        idx = (split + i * BT
               + lax.broadcasted_iota(jnp.int32, (1, BT), 1))
        m = jnp.where(idx < n_rows, m, -jnp.inf)

        @pl.when(i == 0)
        def _():
            bm_ref[...] = jnp.full((1, BT), -jnp.inf, jnp.float32)
            bi_ref[...] = jnp.full((1, BT), IMAX, jnp.int32)

        better = m > bm_ref[...]
        bm_ref[...] = jnp.where(better, m, bm_ref[...])
        bi_ref[...] = jnp.where(better, idx, bi_ref[...])

        @pl.when(i == grid - 1)
        def _():
            bm = bm_ref[...]
            bi = bi_ref[...]
            m_s = jnp.max(bm)
            i_s = jnp.min(jnp.where(bm == m_s, bi, IMAX))
            nm2 = jnp.sum(nm * nm)
            sim = jnp.sign(m_s) * jnp.sqrt(
                jnp.abs(m_s) / jnp.maximum(nm2, EPS2))
            out_ref[0, 0] = m_s
            out_ref[0, 1] = sim
            out_ref[0, 2] = lax.bitcast_convert_type(i_s, jnp.float32)
            out_ref[0, 3] = jnp.where(sim > THRESH, 1.0, 0.0)

    return pl.pallas_call(
        body,
        grid=(grid,),
        in_specs=[
            pl.BlockSpec((DIM, 1), lambda i: (0, 0)),
            pl.BlockSpec((DIM, BT), lambda i: (0, split // BT + i)),
        ],
        out_specs=pl.BlockSpec(memory_space=pltpu.SMEM),
        out_shape=jax.ShapeDtypeStruct((1, 4), jnp.float32),
        scratch_shapes=[
            pltpu.VMEM((1, BT), jnp.float32),
            pltpu.VMEM((1, BT), jnp.int32),
        ],
        compiler_params=pltpu.CompilerParams(
            dimension_semantics=("arbitrary",)),
    )


@jax.jit
def kernel(new_memory, existing_memories):
    n_rows = existing_memories.shape[0]
    emt = existing_memories.T
    split = int(n_rows * 0.295) // BT * BT
    split = max(BT, min(split, n_rows - BT))
    assert split % (128 * NW) == 0 and split % BT == 0

    sim2, m2, idx2, risk2 = _make_sc_call(split)(new_memory, emt)
    tc = _make_tc_call(n_rows, split)(new_memory.reshape(DIM, 1), emt)

    cand_m = jnp.stack([m2[0], m2[L], tc[0, 0]])
    cand_i = jnp.stack([idx2[0], idx2[L],
                        lax.bitcast_convert_type(tc[0, 2], jnp.int32)])
    cand_sim = jnp.stack([sim2[0], sim2[L], tc[0, 1]])
    cand_risk = jnp.stack([risk2[0].astype(jnp.float32),
                           risk2[L].astype(jnp.float32), tc[0, 3]])
    bm, bi, bs, br = cand_m[0], cand_i[0], cand_sim[0], cand_risk[0]
    for k in (1, 2):
        take = (cand_m[k] > bm) | ((cand_m[k] == bm) & (cand_i[k] < bi))
        bm = jnp.where(take, cand_m[k], bm)
        bi = jnp.where(take, cand_i[k], bi)
        bs = jnp.where(take, cand_sim[k], bs)
        br = jnp.where(take, cand_risk[k], br)
    return bs, bi, br != 0.0


# speedup vs baseline: 1.2249x; 1.2249x over previous
import functools

import jax
import jax.numpy as jnp
from jax import lax
from jax.experimental import pallas as pl
from jax.experimental.pallas import tpu as pltpu
from jax.experimental.pallas import tpu_sc as plsc

NC = 2
NS = 16
L = 16
NW = NC * NS

DIM = 64

CHUNK = 256
EPS2 = 1e-16
THRESH = 0.8
IMAX = 2**31 - 1


def _sqrt_vec(x):
    xi = plsc.bitcast(x, jnp.int32)
    r = plsc.bitcast(jnp.int32(0x5F3759DF) - (xi >> 1), jnp.float32)
    for _ in range(4):
        r = r * (1.5 - 0.5 * x * r * r)
    return x * r


def _make_sc_call(n_rows):
    per_w = (n_rows // NW) // 128 * 128
    rem = n_rows - NW * per_w
    rem128 = rem // 128
    rem64 = rem - rem128 * 128
    assert rem64 % L == 0
    full_chunks = per_w // CHUNK
    pairs = full_chunks // 2
    leftover = full_chunks - 2 * pairs
    assert full_chunks >= 2

    mesh = plsc.VectorSubcoreMesh(
        core_axis_name="c", subcore_axis_name="s",
        num_cores=NC, num_subcores=NS)

    lane = lambda: lax.broadcasted_iota(jnp.int32, (L,), 0)

    scratch = [
        pltpu.VMEM((DIM, CHUNK), jnp.float32),
        pltpu.VMEM((DIM, CHUNK), jnp.float32),
        pltpu.VMEM((DIM, max(128, 1)), jnp.float32),
        pltpu.VMEM((DIM, max(rem64, 1)), jnp.float32),
        pltpu.VMEM((DIM,), jnp.float32),
        pltpu.VMEM((L,), jnp.float32),
        pltpu.VMEM((L,), jnp.int32),
        pltpu.VMEM((NS * L,), jnp.float32),
        pltpu.VMEM((NS * L,), jnp.int32),
        pltpu.VMEM((L,), jnp.float32),
        pltpu.VMEM((L,), jnp.int32),
        pltpu.VMEM_SHARED((NS * L,), jnp.float32),
        pltpu.VMEM_SHARED((NS * L,), jnp.int32),
        pltpu.SemaphoreType.DMA,
        pltpu.SemaphoreType.DMA,
        pltpu.SemaphoreType.DMA,
        pltpu.SemaphoreType.DMA,
    ]

    @functools.partial(
        pl.kernel,
        out_type=(
            jax.ShapeDtypeStruct((NC * L,), jnp.float32),
            jax.ShapeDtypeStruct((NC * L,), jnp.float32),
            jax.ShapeDtypeStruct((NC * L,), jnp.int32),
            jax.ShapeDtypeStruct((NC * L,), jnp.int32),
        ),
        mesh=mesh,
        scratch_types=scratch,
        compiler_params=pltpu.CompilerParams(
            needs_layout_passes=False, skip_device_barrier=True),
    )
    def sc_call(nm_hbm, emt_hbm, out_sim, out_m, out_idx, out_risk,
                buf0, buf1, ebuf, fbuf, nm_v, cm_v, ci_v, am_v, ai_v,
                sim_v, risk_v, spm_m, spm_i, sem0, sem1, seme, semf):
        cid = lax.axis_index("c")
        sid = lax.axis_index("s")
        wid = cid * NS + sid
        w_base = wid * per_w + 128 * jnp.minimum(wid, rem128)

        pltpu.sync_copy(nm_hbm, nm_v)
        nm_s = []
        for c0 in range(0, DIM, L):
            v = nm_v[pl.ds(c0, L)]
            nm_s.extend(v[j] for j in range(L))

        def chunk_src(c):
            return emt_hbm.at[:, pl.ds(w_base + c * CHUNK, CHUNK)]

        pltpu.async_copy(chunk_src(0), buf0, sem0)
        pltpu.async_copy(chunk_src(1), buf1, sem1)
        if rem128:
            @pl.when(wid < rem128)
            def _():
                pltpu.async_copy(
                    emt_hbm.at[:, pl.ds(w_base + per_w, 128)], ebuf, seme)
        if rem64:
            @pl.when(wid == NW - 1)
            def _():
                pltpu.async_copy(
                    emt_hbm.at[:, pl.ds(n_rows - rem64, rem64)], fbuf, semf)

        def process(buf, col0, n_groups, best):
            best_m, best_i = best
            d = [jnp.zeros((L,), jnp.float32) for _ in range(n_groups)]
            q = [jnp.zeros((L,), jnp.float32) for _ in range(n_groups)]
            for c in range(DIM):
                s = nm_s[c]
                for g in range(n_groups):
                    e = buf[c, pl.ds(g * L, L)]
                    d[g] = d[g] + e * s
                    q[g] = q[g] + e * e
            for g in range(n_groups):
                m = d[g] * jnp.abs(d[g]) / jnp.maximum(q[g], EPS2)
                idx = col0 + g * L + lane()
                better = m > best_m
                best_m = jnp.where(better, m, best_m)
                best_i = jnp.where(better, idx, best_i)
            return (best_m, best_i)

        best = (jnp.full((L,), -jnp.inf, jnp.float32),
                jnp.full((L,), IMAX, jnp.int32))

        def pair_body(t, best):
            for b, buf, sem in ((0, buf0, sem0), (1, buf1, sem1)):
                c = 2 * t + b
                pltpu.make_async_copy(chunk_src(c), buf, sem).wait()
                best = process(buf, w_base + c * CHUNK, CHUNK // L, best)

                @pl.when(c + 2 <= full_chunks - 1)
                def _():
                    pltpu.async_copy(chunk_src(c + 2), buf, sem)
            return best

        best = lax.fori_loop(0, pairs, pair_body, best)

        if leftover:
            c = full_chunks - 1
            pltpu.make_async_copy(chunk_src(c), buf0, sem0).wait()
            best = process(buf0, w_base + c * CHUNK, CHUNK // L, best)

        if rem128:
            def do_e(best):
                pltpu.make_async_copy(
                    emt_hbm.at[:, pl.ds(w_base + per_w, 128)], ebuf,
                    seme).wait()
                return process(ebuf, w_base + per_w, 128 // L, best)

            best = lax.cond(wid < rem128, do_e, lambda b: b, best)

        if rem64:
            def do_f(best):
                pltpu.make_async_copy(
                    emt_hbm.at[:, pl.ds(n_rows - rem64, rem64)], fbuf,
                    semf).wait()
                return process(fbuf, n_rows - rem64, rem64 // L, best)

            best = lax.cond(wid == NW - 1, do_f, lambda b: b, best)

        best_m, best_i = best

        cm_v[...] = best_m
        ci_v[...] = best_i
        pltpu.sync_copy(cm_v, spm_m.at[pl.ds(sid * L, L)])
        pltpu.sync_copy(ci_v, spm_i.at[pl.ds(sid * L, L)])
        plsc.subcore_barrier()

        @pl.when(sid == 0)
        def _():
            pltpu.sync_copy(spm_m, am_v)
            pltpu.sync_copy(spm_i, ai_v)
            m = am_v[pl.ds(0, L)]
            i = ai_v[pl.ds(0, L)]
            for s in range(1, NS):
                mn = am_v[pl.ds(s * L, L)]
                iN = ai_v[pl.ds(s * L, L)]
                bet = (mn > m) | ((mn == m) & (iN < i))
                m = jnp.where(bet, mn, m)
                i = jnp.where(bet, iN, i)
            m_s = jnp.max(m)
            i_s = jnp.min(jnp.where(m == m_s, i, IMAX))

            nm2_s = jnp.float32(0.0)
            for c0 in range(0, DIM, L):
                v = nm_v[pl.ds(c0, L)]
                nm2_s = nm2_s + jnp.sum(v * v)

            mfin = jnp.full((L,), m_s, jnp.float32)
            s2 = jnp.abs(mfin) / jnp.maximum(
                jnp.full((L,), nm2_s, jnp.float32), EPS2)
            simv = jnp.sign(mfin) * _sqrt_vec(s2)

            sim_v[...] = simv
            cm_v[...] = mfin
            ci_v[...] = jnp.full((L,), i_s, jnp.int32)
            risk_v[...] = jnp.where(simv > THRESH, 1, 0).astype(jnp.int32)
            pltpu.sync_copy(sim_v, out_sim.at[pl.ds(cid * L, L)])
            pltpu.sync_copy(cm_v, out_m.at[pl.ds(cid * L, L)])
            pltpu.sync_copy(ci_v, out_idx.at[pl.ds(cid * L, L)])
            pltpu.sync_copy(risk_v, out_risk.at[pl.ds(cid * L, L)])

    return sc_call


BT = 16384


def _make_tc_call(n_rows, split):
    n_tc = n_rows - split
    grid = (n_tc + BT - 1) // BT
    assert split % BT == 0

    def body(nm_ref, emt_ref, out_ref, bm_ref, bi_ref):
        i = pl.program_id(0)
        e = emt_ref[...]
        nm = nm_ref[...]
        dn = (((0,), (0,)), ((), ()))
        d = lax.dot_general(nm, e, dn,
                            preferred_element_type=jnp.float32)
        q = lax.dot_general(jnp.ones((DIM, 1), jnp.float32), e * e, dn,
                            preferred_element_type=jnp.float32)
        m = d * jnp.abs(d) / jnp.maximum(q, EPS2)
        idx = (split + i * BT
               + lax.broadcasted_iota(jnp.int32, (1, BT), 1))
        m = jnp.where(idx < n_rows, m, -jnp.inf)

        @pl.when(i == 0)
        def _():
            bm_ref[...] = jnp.full((1, BT), -jnp.inf, jnp.float32)
            bi_ref[...] = jnp.full((1, BT), IMAX, jnp.int32)

        better = m > bm_ref[...]
        bm_ref[...] = jnp.where(better, m, bm_ref[...])
        bi_ref[...] = jnp.where(better, idx, bi_ref[...])

        @pl.when(i == grid - 1)
        def _():
            bm = bm_ref[...]
            bi = bi_ref[...]
            m_s = jnp.max(bm)
            i_s = jnp.min(jnp.where(bm == m_s, bi, IMAX))
            nm2 = jnp.sum(nm * nm)
            sim = jnp.sign(m_s) * jnp.sqrt(
                jnp.abs(m_s) / jnp.maximum(nm2, EPS2))
            out_ref[0, 0] = m_s
            out_ref[0, 1] = sim
            out_ref[0, 2] = lax.bitcast_convert_type(i_s, jnp.float32)
            out_ref[0, 3] = jnp.where(sim > THRESH, 1.0, 0.0)

    return pl.pallas_call(
        body,
        grid=(grid,),
        in_specs=[
            pl.BlockSpec((DIM, 1), lambda i: (0, 0)),
            pl.BlockSpec((DIM, BT), lambda i: (0, split // BT + i)),
        ],
        out_specs=pl.BlockSpec(memory_space=pltpu.SMEM),
        out_shape=jax.ShapeDtypeStruct((1, 4), jnp.float32),
        scratch_shapes=[
            pltpu.VMEM((1, BT), jnp.float32),
            pltpu.VMEM((1, BT), jnp.int32),
        ],
        compiler_params=pltpu.CompilerParams(
            dimension_semantics=("arbitrary",)),
    )


@jax.jit
def kernel(new_memory, existing_memories):
    n_rows = existing_memories.shape[0]
    emt = existing_memories.T
    split = int(n_rows * 0.295) // BT * BT
    split = max(BT, min(split, n_rows - BT))
    assert split % (128 * NW) == 0 and split % BT == 0

    sim2, m2, idx2, risk2 = _make_sc_call(split)(new_memory, emt)
    tc = _make_tc_call(n_rows, split)(new_memory.reshape(DIM, 1), emt)

    cand_m = jnp.stack([m2[0], m2[L], tc[0, 0]])
    cand_i = jnp.stack([idx2[0], idx2[L],
                        lax.bitcast_convert_type(tc[0, 2], jnp.int32)])
    cand_sim = jnp.stack([sim2[0], sim2[L], tc[0, 1]])
    cand_risk = jnp.stack([risk2[0].astype(jnp.float32),
                           risk2[L].astype(jnp.float32), tc[0, 3]])
    bm, bi, bs, br = cand_m[0], cand_i[0], cand_sim[0], cand_risk[0]
    for k in (1, 2):
        take = (cand_m[k] > bm) | ((cand_m[k] == bm) & (cand_i[k] < bi))
        bm = jnp.where(take, cand_m[k], bm)
        bi = jnp.where(take, cand_i[k], bi)
        bs = jnp.where(take, cand_sim[k], bs)
        br = jnp.where(take, cand_risk[k], br)
    return bs, bi, br != 0.0


# speedup vs baseline: 1.2372x; 1.0100x over previous
import functools

import jax
import jax.numpy as jnp
from jax import lax
from jax.experimental import pallas as pl
from jax.experimental.pallas import tpu as pltpu
from jax.experimental.pallas import tpu_sc as plsc

NC = 2
NS = 16
L = 16
NW = NC * NS

DIM = 64

CHUNK = 256
EPS2 = 1e-16
THRESH = 0.8
IMAX = 2**31 - 1


def _sqrt_vec(x):
    xi = plsc.bitcast(x, jnp.int32)
    r = plsc.bitcast(jnp.int32(0x5F3759DF) - (xi >> 1), jnp.float32)
    for _ in range(4):
        r = r * (1.5 - 0.5 * x * r * r)
    return x * r


def _make_sc_call(n_rows):
    per_w = (n_rows // NW) // 128 * 128
    rem = n_rows - NW * per_w
    rem128 = rem // 128
    rem64 = rem - rem128 * 128
    assert rem64 % L == 0
    full_chunks = per_w // CHUNK
    pairs = full_chunks // 2
    leftover = full_chunks - 2 * pairs
    assert full_chunks >= 2

    mesh = plsc.VectorSubcoreMesh(
        core_axis_name="c", subcore_axis_name="s",
        num_cores=NC, num_subcores=NS)

    lane = lambda: lax.broadcasted_iota(jnp.int32, (L,), 0)

    scratch = [
        pltpu.VMEM((DIM, CHUNK), jnp.float32),
        pltpu.VMEM((DIM, CHUNK), jnp.float32),
        pltpu.VMEM((DIM, max(128, 1)), jnp.float32),
        pltpu.VMEM((DIM, max(rem64, 1)), jnp.float32),
        pltpu.VMEM((DIM,), jnp.float32),
        pltpu.VMEM((L,), jnp.float32),
        pltpu.VMEM((L,), jnp.int32),
        pltpu.VMEM((NS * L,), jnp.float32),
        pltpu.VMEM((NS * L,), jnp.int32),
        pltpu.VMEM((L,), jnp.float32),
        pltpu.VMEM((L,), jnp.int32),
        pltpu.VMEM_SHARED((NS * L,), jnp.float32),
        pltpu.VMEM_SHARED((NS * L,), jnp.int32),
        pltpu.SemaphoreType.DMA,
        pltpu.SemaphoreType.DMA,
        pltpu.SemaphoreType.DMA,
        pltpu.SemaphoreType.DMA,
    ]

    @functools.partial(
        pl.kernel,
        out_type=(
            jax.ShapeDtypeStruct((NC * L,), jnp.float32),
            jax.ShapeDtypeStruct((NC * L,), jnp.float32),
            jax.ShapeDtypeStruct((NC * L,), jnp.int32),
            jax.ShapeDtypeStruct((NC * L,), jnp.int32),
        ),
        mesh=mesh,
        scratch_types=scratch,
        compiler_params=pltpu.CompilerParams(
            needs_layout_passes=False, skip_device_barrier=True),
    )
    def sc_call(nm_hbm, emt_hbm, out_sim, out_m, out_idx, out_risk,
                buf0, buf1, ebuf, fbuf, nm_v, cm_v, ci_v, am_v, ai_v,
                sim_v, risk_v, spm_m, spm_i, sem0, sem1, seme, semf):
        cid = lax.axis_index("c")
        sid = lax.axis_index("s")
        wid = cid * NS + sid
        w_base = wid * per_w + 128 * jnp.minimum(wid, rem128)

        pltpu.sync_copy(nm_hbm, nm_v)
        nm_s = []
        for c0 in range(0, DIM, L):
            v = nm_v[pl.ds(c0, L)]
            nm_s.extend(v[j] for j in range(L))

        def chunk_src(c):
            return emt_hbm.at[:, pl.ds(w_base + c * CHUNK, CHUNK)]

        pltpu.async_copy(chunk_src(0), buf0, sem0)
        pltpu.async_copy(chunk_src(1), buf1, sem1)
        if rem128:
            @pl.when(wid < rem128)
            def _():
                pltpu.async_copy(
                    emt_hbm.at[:, pl.ds(w_base + per_w, 128)], ebuf, seme)
        if rem64:
            @pl.when(wid == NW - 1)
            def _():
                pltpu.async_copy(
                    emt_hbm.at[:, pl.ds(n_rows - rem64, rem64)], fbuf, semf)

        def process(buf, col0, n_groups, best):
            best_m, best_i = best
            d = [jnp.zeros((L,), jnp.float32) for _ in range(n_groups)]
            q = [jnp.zeros((L,), jnp.float32) for _ in range(n_groups)]
            for c in range(DIM):
                s = nm_s[c]
                for g in range(n_groups):
                    e = buf[c, pl.ds(g * L, L)]
                    d[g] = d[g] + e * s
                    q[g] = q[g] + e * e
            for g in range(n_groups):
                m = d[g] * jnp.abs(d[g]) / jnp.maximum(q[g], EPS2)
                idx = col0 + g * L + lane()
                better = m > best_m
                best_m = jnp.where(better, m, best_m)
                best_i = jnp.where(better, idx, best_i)
            return (best_m, best_i)

        best = (jnp.full((L,), -jnp.inf, jnp.float32),
                jnp.full((L,), IMAX, jnp.int32))

        def pair_body(t, best):
            for b, buf, sem in ((0, buf0, sem0), (1, buf1, sem1)):
                c = 2 * t + b
                pltpu.make_async_copy(chunk_src(c), buf, sem).wait()
                best = process(buf, w_base + c * CHUNK, CHUNK // L, best)

                @pl.when(c + 2 <= full_chunks - 1)
                def _():
                    pltpu.async_copy(chunk_src(c + 2), buf, sem)
            return best

        best = lax.fori_loop(0, pairs, pair_body, best)

        if leftover:
            c = full_chunks - 1
            pltpu.make_async_copy(chunk_src(c), buf0, sem0).wait()
            best = process(buf0, w_base + c * CHUNK, CHUNK // L, best)

        if rem128:
            def do_e(best):
                pltpu.make_async_copy(
                    emt_hbm.at[:, pl.ds(w_base + per_w, 128)], ebuf,
                    seme).wait()
                return process(ebuf, w_base + per_w, 128 // L, best)

            best = lax.cond(wid < rem128, do_e, lambda b: b, best)

        if rem64:
            def do_f(best):
                pltpu.make_async_copy(
                    emt_hbm.at[:, pl.ds(n_rows - rem64, rem64)], fbuf,
                    semf).wait()
                return process(fbuf, n_rows - rem64, rem64 // L, best)

            best = lax.cond(wid == NW - 1, do_f, lambda b: b, best)

        best_m, best_i = best

        cm_v[...] = best_m
        ci_v[...] = best_i
        pltpu.sync_copy(cm_v, spm_m.at[pl.ds(sid * L, L)])
        pltpu.sync_copy(ci_v, spm_i.at[pl.ds(sid * L, L)])
        plsc.subcore_barrier()

        @pl.when(sid == 0)
        def _():
            pltpu.sync_copy(spm_m, am_v)
            pltpu.sync_copy(spm_i, ai_v)
            m = am_v[pl.ds(0, L)]
            i = ai_v[pl.ds(0, L)]
            for s in range(1, NS):
                mn = am_v[pl.ds(s * L, L)]
                iN = ai_v[pl.ds(s * L, L)]
                bet = (mn > m) | ((mn == m) & (iN < i))
                m = jnp.where(bet, mn, m)
                i = jnp.where(bet, iN, i)
            m_s = jnp.max(m)
            i_s = jnp.min(jnp.where(m == m_s, i, IMAX))

            nm2_s = jnp.float32(0.0)
            for c0 in range(0, DIM, L):
                v = nm_v[pl.ds(c0, L)]
                nm2_s = nm2_s + jnp.sum(v * v)

            mfin = jnp.full((L,), m_s, jnp.float32)
            s2 = jnp.abs(mfin) / jnp.maximum(
                jnp.full((L,), nm2_s, jnp.float32), EPS2)
            simv = jnp.sign(mfin) * _sqrt_vec(s2)

            sim_v[...] = simv
            cm_v[...] = mfin
            ci_v[...] = jnp.full((L,), i_s, jnp.int32)
            risk_v[...] = jnp.where(simv > THRESH, 1, 0).astype(jnp.int32)
            pltpu.sync_copy(sim_v, out_sim.at[pl.ds(cid * L, L)])
            pltpu.sync_copy(cm_v, out_m.at[pl.ds(cid * L, L)])
            pltpu.sync_copy(ci_v, out_idx.at[pl.ds(cid * L, L)])
            pltpu.sync_copy(risk_v, out_risk.at[pl.ds(cid * L, L)])

    return sc_call


BT = 16384


def _make_tc_call(n_rows, split):
    n_tc = n_rows - split
    grid = (n_tc + BT - 1) // BT
    assert split % BT == 0

    def body(nm_ref, emt_ref, out_ref, bm_ref, bi_ref):
        i = pl.program_id(0)
        e = emt_ref[...]
        nm = nm_ref[...]
        d = jnp.sum(e * nm, axis=0, keepdims=True)
        q = jnp.sum(e * e, axis=0, keepdims=True)
        m = d * jnp.abs(d) / jnp.maximum(q, EPS2)
        idx = (split + i * BT
               + lax.broadcasted_iota(jnp.int32, (1, BT), 1))
        m = jnp.where(idx < n_rows, m, -jnp.inf)

        @pl.when(i == 0)
        def _():
            bm_ref[...] = jnp.full((1, BT), -jnp.inf, jnp.float32)
            bi_ref[...] = jnp.full((1, BT), IMAX, jnp.int32)

        better = m > bm_ref[...]
        bm_ref[...] = jnp.where(better, m, bm_ref[...])
        bi_ref[...] = jnp.where(better, idx, bi_ref[...])

        @pl.when(i == grid - 1)
        def _():
            bm = bm_ref[...]
            bi = bi_ref[...]
            m_s = jnp.max(bm)
            i_s = jnp.min(jnp.where(bm == m_s, bi, IMAX))
            nm2 = jnp.sum(nm * nm)
            sim = jnp.sign(m_s) * jnp.sqrt(
                jnp.abs(m_s) / jnp.maximum(nm2, EPS2))
            out_ref[0, 0] = m_s
            out_ref[0, 1] = sim
            out_ref[0, 2] = lax.bitcast_convert_type(i_s, jnp.float32)
            out_ref[0, 3] = jnp.where(sim > THRESH, 1.0, 0.0)

    return pl.pallas_call(
        body,
        grid=(grid,),
        in_specs=[
            pl.BlockSpec((DIM, 1), lambda i: (0, 0)),
            pl.BlockSpec((DIM, BT), lambda i: (0, split // BT + i)),
        ],
        out_specs=pl.BlockSpec(memory_space=pltpu.SMEM),
        out_shape=jax.ShapeDtypeStruct((1, 4), jnp.float32),
        scratch_shapes=[
            pltpu.VMEM((1, BT), jnp.float32),
            pltpu.VMEM((1, BT), jnp.int32),
        ],
        compiler_params=pltpu.CompilerParams(
            dimension_semantics=("arbitrary",)),
    )


@jax.jit
def kernel(new_memory, existing_memories):
    n_rows = existing_memories.shape[0]
    emt = existing_memories.T
    split = int(n_rows * 0.28) // BT * BT
    split = max(BT, min(split, n_rows - BT))
    assert split % (128 * NW) == 0 and split % BT == 0

    sim2, m2, idx2, risk2 = _make_sc_call(split)(new_memory, emt)
    tc = _make_tc_call(n_rows, split)(new_memory.reshape(DIM, 1), emt)

    cand_m = jnp.stack([m2[0], m2[L], tc[0, 0]])
    cand_i = jnp.stack([idx2[0], idx2[L],
                        lax.bitcast_convert_type(tc[0, 2], jnp.int32)])
    cand_sim = jnp.stack([sim2[0], sim2[L], tc[0, 1]])
    cand_risk = jnp.stack([risk2[0].astype(jnp.float32),
                           risk2[L].astype(jnp.float32), tc[0, 3]])
    bm, bi, bs, br = cand_m[0], cand_i[0], cand_sim[0], cand_risk[0]
    for k in (1, 2):
        take = (cand_m[k] > bm) | ((cand_m[k] == bm) & (cand_i[k] < bi))
        bm = jnp.where(take, cand_m[k], bm)
        bi = jnp.where(take, cand_i[k], bi)
        bs = jnp.where(take, cand_sim[k], bs)
        br = jnp.where(take, cand_risk[k], br)
    return bs, bi, br != 0.0
